# baseline (speedup 1.0000x reference)
import functools

import jax
import jax.numpy as jnp
import numpy as np
from jax import lax
from jax.experimental import pallas as pl
from jax.experimental.pallas import tpu as pltpu
from jax.experimental.pallas import tpu_sc as plsc

_SC_CORES = 2
_SC_SUBCORES = 16
_SC_WORKERS = _SC_CORES * _SC_SUBCORES
_GATHER_CHUNK = 64

_BLK = 256


def _router(x, w_gate):
    T, Dm = x.shape
    E = w_gate.shape[1]
    TB = 256

    def body(x_ref, wg_ref, i1_ref, i2_ref, g1_ref, g2_ref):
        logits = lax.dot_general(
            x_ref[...], wg_ref[...], (((1,), (0,)), ((), ())),
            preferred_element_type=jnp.float32)
        iota = lax.broadcasted_iota(jnp.int32, logits.shape, 1)
        v1 = jnp.max(logits, axis=1, keepdims=True)
        i1 = jnp.min(jnp.where(logits == v1, iota, E), axis=1, keepdims=True)
        masked = jnp.where(iota == i1, -jnp.inf, logits)
        v2 = jnp.max(masked, axis=1, keepdims=True)
        i2 = jnp.min(
            jnp.where((masked == v2) & (iota != i1), iota, E),
            axis=1, keepdims=True)
        e2 = jnp.exp(v2 - v1)
        den = 1.0 + e2
        i1_ref[...] = i1
        i2_ref[...] = i2
        g1_ref[...] = 1.0 / den
        g2_ref[...] = e2 / den

    return pl.pallas_call(
        body,
        grid=(T // TB,),
        in_specs=[
            pl.BlockSpec((TB, Dm), lambda i: (i, 0)),
            pl.BlockSpec((Dm, E), lambda i: (0, 0)),
        ],
        out_specs=[pl.BlockSpec((TB, 1), lambda i: (i, 0))] * 4,
        out_shape=[
            jax.ShapeDtypeStruct((T, 1), jnp.int32),
            jax.ShapeDtypeStruct((T, 1), jnp.int32),
            jax.ShapeDtypeStruct((T, 1), jnp.float32),
            jax.ShapeDtypeStruct((T, 1), jnp.float32),
        ],
    )(x, w_gate)


def _sc_gather_rows(table, idx):
    N, D = table.shape
    B = idx.shape[0]
    per_w = B // _SC_WORKERS
    n_ch = per_w // _GATHER_CHUNK
    mesh = plsc.VectorSubcoreMesh(core_axis_name="c", subcore_axis_name="s")

    @functools.partial(
        pl.kernel,
        mesh=mesh,
        out_type=jax.ShapeDtypeStruct((B, D), table.dtype),
        scratch_types=[
            pltpu.VMEM((_GATHER_CHUNK,), jnp.int32),
            pltpu.VMEM((_GATHER_CHUNK, D), table.dtype),
            pltpu.SemaphoreType.DMA,
        ],
    )
    def gather_kernel(table_hbm, idx_hbm, out_hbm, idx_v, rows_v, sem):
        wid = lax.axis_index("s") * _SC_CORES + lax.axis_index("c")
        base = wid * per_w

        def chunk(c, carry):
            off = base + c * _GATHER_CHUNK
            pltpu.sync_copy(idx_hbm.at[pl.ds(off, _GATHER_CHUNK)], idx_v)
            pltpu.async_copy(table_hbm.at[idx_v], rows_v, sem).wait()
            pltpu.sync_copy(rows_v, out_hbm.at[pl.ds(off, _GATHER_CHUNK)])
            return carry

        lax.fori_loop(0, n_ch, chunk, 0)

    return gather_kernel(table, idx)


def _expert_blocks(x_sorted, w1b, w2b, row_gate, blk_exp):
    P, Dm = x_sorted.shape
    E, _, F = w1b.shape
    R = P // _BLK

    def body(be_ref, x_ref, w1_ref, w2_ref, g_ref, z_ref):
        xb = x_ref[...].astype(jnp.bfloat16)
        h = lax.dot_general(
            xb, w1_ref[0], (((1,), (0,)), ((), ())),
            preferred_element_type=jnp.float32)
        hb = jnp.maximum(h, 0.0).astype(jnp.bfloat16)
        y = lax.dot_general(
            hb, w2_ref[0], (((1,), (0,)), ((), ())),
            preferred_element_type=jnp.float32)
        z_ref[...] = g_ref[...] * jnp.exp(y)

    grid_spec = pltpu.PrefetchScalarGridSpec(
        num_scalar_prefetch=1,
        grid=(R,),
        in_specs=[
            pl.BlockSpec((_BLK, Dm), lambda i, be: (i, 0)),
            pl.BlockSpec((1, Dm, F), lambda i, be: (be[i], 0, 0)),
            pl.BlockSpec((1, F, Dm), lambda i, be: (be[i], 0, 0)),
            pl.BlockSpec((_BLK, 1), lambda i, be: (i, 0)),
        ],
        out_specs=pl.BlockSpec((_BLK, Dm), lambda i, be: (i, 0)),
    )
    return pl.pallas_call(
        body,
        grid_spec=grid_spec,
        out_shape=jax.ShapeDtypeStruct((P, Dm), jnp.float32),
        compiler_params=pltpu.CompilerParams(
            dimension_semantics=("arbitrary",)),
    )(blk_exp, x_sorted, w1b, w2b, row_gate)


def _combine(zz):
    _, T, Dm = zz.shape
    TB = 256
    eps = np.float32(np.finfo(np.float64).eps)

    def body(zz_ref, o_ref):
        s = zz_ref[0] + zz_ref[1]
        o_ref[...] = jnp.log(jnp.where(s == 0.0, eps, s))

    return pl.pallas_call(
        body,
        grid=(T // TB,),
        in_specs=[pl.BlockSpec((2, TB, Dm), lambda i: (0, i, 0))],
        out_specs=pl.BlockSpec((TB, Dm), lambda i: (i, 0)),
        out_shape=jax.ShapeDtypeStruct((T, Dm), jnp.float32),
    )(zz)


def kernel(x, w_gate, w1, w2):
    T, Dm = x.shape
    E = w_gate.shape[1]
    R = (2 * T) // _BLK + E - 1
    R += (-R) % 8
    P = R * _BLK

    i1, i2, g1, g2 = _router(x, w_gate)

    i32 = jnp.int32
    e_all = jnp.concatenate([i1[:, 0], i2[:, 0]])
    g_all = jnp.concatenate([g1[:, 0], g2[:, 0]])
    ar = jnp.arange(T, dtype=i32)
    tok_all = jnp.concatenate([ar, ar])
    oh = (e_all[:, None] == jnp.arange(E, dtype=i32)[None, :]).astype(i32)
    csum = jnp.cumsum(oh, axis=0)
    counts = csum[-1]
    nblk = (counts + _BLK - 1) // _BLK
    blk_start = jnp.concatenate(
        [jnp.zeros((1,), i32), jnp.cumsum(nblk)[:-1].astype(i32)])
    rank = jnp.take_along_axis(csum, e_all[:, None], axis=1)[:, 0] - 1
    dst = blk_start[e_all] * _BLK + rank
    row_token = jnp.zeros((P,), i32).at[dst].set(tok_all)
    row_gate = jnp.zeros((P, 1), jnp.float32).at[dst, 0].set(g_all)
    blk_exp = jnp.clip(
        (jnp.arange(R, dtype=i32)[:, None] >= blk_start[None, :]).sum(
            axis=1) - 1, 0, E - 1).astype(i32)

    x_sorted = _sc_gather_rows(x, row_token)

    w1b = w1.astype(jnp.bfloat16)
    w2b = w2.astype(jnp.bfloat16)
    z = _expert_blocks(x_sorted, w1b, w2b, row_gate, blk_exp)

    zz = _sc_gather_rows(z, dst).reshape(2, T, Dm)

    return _combine(zz)
